# baseline (speedup 1.0000x reference)
import functools
import jax
import jax.numpy as jnp
from jax import lax
from jax.experimental import pallas as pl

_B, _D = 4, 128
_NL, _NP, _NG = 32768, 8192, 4096
_N = _NL + _NP + _NG
_K = 1024


def _score_body(src_id, sfull_ref, q_ref, stile_ref, w1a_ref, wrow_ref,
                b1_ref, erow_ref, w2_ref, b2_ref, keep_ref, snorm_ref):
    inf = jnp.float32(jnp.inf)
    s_row = sfull_ref[0, 0, :]
    if src_id == 0:
        st_row = jnp.log1p(s_row)
        valid_row = s_row > 0
    elif src_id == 1:
        st_row = s_row
        valid_row = s_row > 0
    else:
        st_row = s_row
        valid_row = jnp.ones_like(s_row, dtype=jnp.bool_)
    mn = jnp.min(jnp.where(valid_row, st_row, inf))
    mx = jnp.max(jnp.where(valid_row, st_row, -inf))
    any_valid = jnp.any(valid_row)
    rng = mx - mn
    degen = jnp.abs(rng) < 1e-6
    denom = jnp.where(degen, jnp.float32(1.0), rng)

    s_t = stile_ref[0, 0, :]
    if src_id == 0:
        st_t = jnp.log1p(s_t)
        valid_t = s_t > 0
    elif src_id == 1:
        st_t = s_t
        valid_t = s_t > 0
    else:
        st_t = s_t
        valid_t = jnp.ones_like(s_t, dtype=jnp.bool_)
    val = jnp.where(degen, jnp.float32(1.0), (st_t - mn) / denom)
    s_norm = jnp.where(valid_t & any_valid, val, jnp.float32(0.0))

    bf16 = jnp.bfloat16
    x = q_ref[0]
    xb = (x + erow_ref[0]).astype(bf16)
    sb = s_norm.astype(bf16)
    zx = jnp.zeros((xb.shape[0], 127), bf16)
    xcat = jnp.concatenate([xb, sb[:, None], zx], axis=1)
    w1b = w1a_ref[...].astype(bf16)
    wrowb = wrow_ref[0].astype(bf16)
    zw = jnp.zeros((127, _D), bf16)
    wcat = jnp.concatenate([w1b, wrowb[None, :], zw], axis=0)
    h = jnp.dot(xcat, wcat, preferred_element_type=jnp.float32)
    h = h + b1_ref[0]
    h = jnp.maximum(h, jnp.float32(0.0))
    hb = h.astype(bf16)
    w2b = w2_ref[...].astype(bf16)
    logit = jnp.dot(hb, w2b, preferred_element_type=jnp.float32)[:, 0]
    keep = logit + b2_ref[0, 0] + s_norm
    keep_ref[0, 0, :] = jnp.where(valid_t, keep, -inf)
    snorm_ref[0, 0, :] = s_norm


def _make_scorer(src_id, n_src, tile):
    grid = (_B, n_src // tile)
    f32 = jnp.float32
    return pl.pallas_call(
        functools.partial(_score_body, src_id),
        grid=grid,
        in_specs=[
            pl.BlockSpec((1, 1, n_src), lambda b, t: (b, 0, 0)),
            pl.BlockSpec((1, tile, _D), lambda b, t: (b, t, 0)),
            pl.BlockSpec((1, 1, tile), lambda b, t: (b, 0, t)),
            pl.BlockSpec((_D, _D), lambda b, t: (0, 0)),
            pl.BlockSpec((1, _D), lambda b, t: (0, 0)),
            pl.BlockSpec((1, _D), lambda b, t: (0, 0)),
            pl.BlockSpec((1, _D), lambda b, t: (0, 0)),
            pl.BlockSpec((_D, 1), lambda b, t: (0, 0)),
            pl.BlockSpec((1, 1), lambda b, t: (0, 0)),
        ],
        out_specs=[
            pl.BlockSpec((1, 1, tile), lambda b, t: (b, 0, t)),
            pl.BlockSpec((1, 1, tile), lambda b, t: (b, 0, t)),
        ],
        out_shape=[
            jax.ShapeDtypeStruct((_B, 1, n_src), f32),
            jax.ShapeDtypeStruct((_B, 1, n_src), f32),
        ],
    )


def kernel(lidar_queries, lidar_refs, lidar_scores, lidar_prior_labels,
           lidar_prior_scores, lidar_prior_valid_mask, proposal_queries,
           proposal_refs, proposal_scores, global_queries, global_refs,
           global_scores, source_embeddings, W1, b1, W2, b2):
    f32 = jnp.float32
    w1a = W1[:_D]
    wrow = W1[_D:_D + 1]
    b1r = b1[None, :]
    b2r = b2[None, :]

    ls3 = lidar_scores[:, None, :]
    ps3 = proposal_scores[:, None, :]
    gs3 = global_scores[:, None, :]
    keep_l, sn_l = _make_scorer(0, _NL, 2048)(
        ls3, lidar_queries, ls3, w1a, wrow, b1r,
        source_embeddings[0:1], W2, b2r)
    keep_p, sn_p = _make_scorer(1, _NP, 2048)(
        ps3, proposal_queries, ps3, w1a, wrow, b1r,
        source_embeddings[1:2], W2, b2r)
    keep_g, sn_g = _make_scorer(2, _NG, 2048)(
        gs3, global_queries, gs3, w1a, wrow, b1r,
        source_embeddings[2:3], W2, b2r)

    keep = jnp.concatenate([keep_l[:, 0], keep_p[:, 0], keep_g[:, 0]], axis=1)
    snorm = jnp.concatenate([sn_l[:, 0], sn_p[:, 0], sn_g[:, 0]], axis=1)

    top_vals, top_idx = lax.top_k(keep, _K)

    routed_scores = jnp.take_along_axis(snorm, top_idx, axis=1)
    in_l = top_idx < _NL
    in_p = (top_idx >= _NL) & (top_idx < _NL + _NP)
    routed_src = in_l.astype(jnp.int32) * 0 + jnp.where(
        in_l, 0, jnp.where(in_p, 1, 2)).astype(jnp.int32)

    idx_l = jnp.clip(top_idx, 0, _NL - 1)
    idx_p = jnp.clip(top_idx - _NL, 0, _NP - 1)
    idx_g = jnp.clip(top_idx - _NL - _NP, 0, _NG - 1)

    q_l = jnp.take_along_axis(lidar_queries, idx_l[..., None], axis=1)
    q_p = jnp.take_along_axis(proposal_queries, idx_p[..., None], axis=1)
    q_g = jnp.take_along_axis(global_queries, idx_g[..., None], axis=1)
    routed_q = jnp.where(
        in_l[..., None], q_l + source_embeddings[0],
        jnp.where(in_p[..., None], q_p + source_embeddings[1],
                  q_g + source_embeddings[2]))

    r_l = jnp.take_along_axis(lidar_refs, idx_l[..., None], axis=1)
    r_p = jnp.take_along_axis(proposal_refs, idx_p[..., None], axis=1)
    r_g = jnp.take_along_axis(global_refs, idx_g[..., None], axis=1)
    routed_refs = jnp.where(in_l[..., None], r_l,
                            jnp.where(in_p[..., None], r_p, r_g))

    routed_pl = jnp.where(
        in_l, jnp.take_along_axis(lidar_prior_labels, idx_l, axis=1),
        jnp.zeros((), lidar_prior_labels.dtype))
    routed_psc = jnp.where(
        in_l, jnp.take_along_axis(lidar_prior_scores, idx_l, axis=1),
        jnp.zeros((), f32))
    routed_pvalid = jnp.where(
        in_l, jnp.take_along_axis(lidar_prior_valid_mask, idx_l, axis=1),
        False)

    return (routed_q, routed_refs, routed_scores, routed_src, routed_pl,
            routed_psc, routed_pvalid, top_vals)


# speedup vs baseline: 1.1857x; 1.1857x over previous
import functools
import jax
import jax.numpy as jnp
from jax import lax
from jax.experimental import pallas as pl

_B, _D = 4, 128
_NL, _NP, _NG = 32768, 8192, 4096
_N = _NL + _NP + _NG
_K = 1024


def _score_body(src_id, sfull_ref, q_ref, stile_ref, w1a_ref, wrow_ref,
                b1_ref, erow_ref, w2_ref, b2_ref, keep_ref, snorm_ref):
    inf = jnp.float32(jnp.inf)
    s_row = sfull_ref[0, 0, :]
    if src_id == 0:
        st_row = jnp.log1p(s_row)
        valid_row = s_row > 0
    elif src_id == 1:
        st_row = s_row
        valid_row = s_row > 0
    else:
        st_row = s_row
        valid_row = jnp.ones_like(s_row, dtype=jnp.bool_)
    mn = jnp.min(jnp.where(valid_row, st_row, inf))
    mx = jnp.max(jnp.where(valid_row, st_row, -inf))
    any_valid = jnp.any(valid_row)
    rng = mx - mn
    degen = jnp.abs(rng) < 1e-6
    denom = jnp.where(degen, jnp.float32(1.0), rng)

    s_t = stile_ref[0, 0, :]
    if src_id == 0:
        st_t = jnp.log1p(s_t)
        valid_t = s_t > 0
    elif src_id == 1:
        st_t = s_t
        valid_t = s_t > 0
    else:
        st_t = s_t
        valid_t = jnp.ones_like(s_t, dtype=jnp.bool_)
    val = jnp.where(degen, jnp.float32(1.0), (st_t - mn) / denom)
    s_norm = jnp.where(valid_t & any_valid, val, jnp.float32(0.0))

    bf16 = jnp.bfloat16
    x = q_ref[0]
    xb = (x + erow_ref[0]).astype(bf16)
    sb = s_norm.astype(bf16)
    zx = jnp.zeros((xb.shape[0], 127), bf16)
    xcat = jnp.concatenate([xb, sb[:, None], zx], axis=1)
    w1b = w1a_ref[...].astype(bf16)
    wrowb = wrow_ref[0].astype(bf16)
    zw = jnp.zeros((127, _D), bf16)
    wcat = jnp.concatenate([w1b, wrowb[None, :], zw], axis=0)
    h = jnp.dot(xcat, wcat, preferred_element_type=jnp.float32)
    h = h + b1_ref[0]
    h = jnp.maximum(h, jnp.float32(0.0))
    hb = h.astype(bf16)
    w2b = w2_ref[...].astype(bf16)
    logit = jnp.dot(hb, w2b, preferred_element_type=jnp.float32)[:, 0]
    keep = logit + b2_ref[0, 0] + s_norm
    keep_ref[0, 0, :] = jnp.where(valid_t, keep, -inf)
    snorm_ref[0, 0, :] = s_norm


def _make_scorer(src_id, n_src, tile):
    grid = (_B, n_src // tile)
    f32 = jnp.float32
    return pl.pallas_call(
        functools.partial(_score_body, src_id),
        grid=grid,
        in_specs=[
            pl.BlockSpec((1, 1, n_src), lambda b, t: (b, 0, 0)),
            pl.BlockSpec((1, tile, _D), lambda b, t: (b, t, 0)),
            pl.BlockSpec((1, 1, tile), lambda b, t: (b, 0, t)),
            pl.BlockSpec((_D, _D), lambda b, t: (0, 0)),
            pl.BlockSpec((1, _D), lambda b, t: (0, 0)),
            pl.BlockSpec((1, _D), lambda b, t: (0, 0)),
            pl.BlockSpec((1, _D), lambda b, t: (0, 0)),
            pl.BlockSpec((_D, 1), lambda b, t: (0, 0)),
            pl.BlockSpec((1, 1), lambda b, t: (0, 0)),
        ],
        out_specs=[
            pl.BlockSpec((1, 1, tile), lambda b, t: (b, 0, t)),
            pl.BlockSpec((1, 1, tile), lambda b, t: (b, 0, t)),
        ],
        out_shape=[
            jax.ShapeDtypeStruct((_B, 1, n_src), f32),
            jax.ShapeDtypeStruct((_B, 1, n_src), f32),
        ],
    )


def kernel(lidar_queries, lidar_refs, lidar_scores, lidar_prior_labels,
           lidar_prior_scores, lidar_prior_valid_mask, proposal_queries,
           proposal_refs, proposal_scores, global_queries, global_refs,
           global_scores, source_embeddings, W1, b1, W2, b2):
    f32 = jnp.float32
    w1a = W1[:_D]
    wrow = W1[_D:_D + 1]
    b1r = b1[None, :]
    b2r = b2[None, :]

    ls3 = lidar_scores[:, None, :]
    ps3 = proposal_scores[:, None, :]
    gs3 = global_scores[:, None, :]
    keep_l, sn_l = _make_scorer(0, _NL, 2048)(
        ls3, lidar_queries, ls3, w1a, wrow, b1r,
        source_embeddings[0:1], W2, b2r)
    keep_p, sn_p = _make_scorer(1, _NP, 2048)(
        ps3, proposal_queries, ps3, w1a, wrow, b1r,
        source_embeddings[1:2], W2, b2r)
    keep_g, sn_g = _make_scorer(2, _NG, 2048)(
        gs3, global_queries, gs3, w1a, wrow, b1r,
        source_embeddings[2:3], W2, b2r)

    keep = jnp.concatenate([keep_l[:, 0], keep_p[:, 0], keep_g[:, 0]], axis=1)

    top_vals, top_idx = lax.top_k(keep, _K)

    fl = jnp.concatenate([
        lidar_refs, sn_l[:, 0][..., None], lidar_prior_scores[..., None],
        lidar_prior_labels.astype(f32)[..., None],
        lidar_prior_valid_mask.astype(f32)[..., None]], axis=-1)
    fp = jnp.concatenate([
        proposal_refs, sn_p[:, 0][..., None],
        jnp.zeros((_B, _NP, 3), f32)], axis=-1)
    fg = jnp.concatenate([
        global_refs, sn_g[:, 0][..., None],
        jnp.zeros((_B, _NG, 3), f32)], axis=-1)
    fields = jnp.concatenate([fl, fp, fg], axis=1)
    g = jnp.take_along_axis(fields, top_idx[..., None], axis=1)
    routed_refs = g[..., 0:3]
    routed_scores = g[..., 3]
    routed_psc = g[..., 4]
    routed_pl = g[..., 5].astype(lidar_prior_labels.dtype)
    routed_pvalid = g[..., 6] != 0

    in_l = top_idx < _NL
    in_p = (top_idx >= _NL) & (top_idx < _NL + _NP)
    routed_src = jnp.where(in_l, 0, jnp.where(in_p, 1, 2)).astype(jnp.int32)

    idx_l = jnp.clip(top_idx, 0, _NL - 1)
    idx_p = jnp.clip(top_idx - _NL, 0, _NP - 1)
    idx_g = jnp.clip(top_idx - _NL - _NP, 0, _NG - 1)

    q_l = jnp.take_along_axis(lidar_queries, idx_l[..., None], axis=1)
    q_p = jnp.take_along_axis(proposal_queries, idx_p[..., None], axis=1)
    q_g = jnp.take_along_axis(global_queries, idx_g[..., None], axis=1)
    routed_q = jnp.where(
        in_l[..., None], q_l + source_embeddings[0],
        jnp.where(in_p[..., None], q_p + source_embeddings[1],
                  q_g + source_embeddings[2]))

    return (routed_q, routed_refs, routed_scores, routed_src, routed_pl,
            routed_psc, routed_pvalid, top_vals)
